# Initial kernel scaffold; baseline (speedup 1.0000x reference)
import jax
import jax.numpy as jnp
from jax.experimental import pallas as pl


def kernel(input_ids, word_embeddings):
    raise NotImplementedError("write your pallas kernel here")


# speedup vs baseline: 1.0174x; 1.0174x over previous
import functools

import jax
import jax.numpy as jnp
from jax import lax
from jax.experimental import pallas as pl
from jax.experimental.pallas import tpu as pltpu
from jax.experimental.pallas import tpu_sc as plsc

_NC = 2
_NS = 16
_NW = _NC * _NS

_K = 8


@functools.lru_cache(maxsize=None)
def _build_sc_gather(b_tot, v, h, k, nch):
    mesh = plsc.VectorSubcoreMesh(
        core_axis_name="c", subcore_axis_name="s",
        num_cores=_NC, num_subcores=_NS,
    )

    @functools.partial(
        pl.kernel,
        out_type=jax.ShapeDtypeStruct((b_tot, h), jnp.float32),
        mesh=mesh,
        scratch_types=[
            pltpu.VMEM((nch, k), jnp.int32),
            pltpu.VMEM((2, k, h), jnp.float32),
            pltpu.SemaphoreType.DMA,
            pltpu.SemaphoreType.DMA,
            pltpu.SemaphoreType.DMA,
            pltpu.SemaphoreType.DMA,
        ],
    )
    def body(idx_hbm, table_hbm, out_hbm, idx_v, buf, gsem0, gsem1,
             wsem0, wsem1):
        wid = lax.axis_index("s") * _NC + lax.axis_index("c")
        base = wid * (nch * k)
        pltpu.sync_copy(idx_hbm.at[wid], idx_v)

        gsems = (gsem0, gsem1)
        wsems = (wsem0, wsem1)

        def gather_start(j, p):
            pltpu.async_copy(table_hbm.at[idx_v.at[j]], buf.at[p], gsems[p])

        def gather_wait(j, p):
            pltpu.make_async_copy(
                table_hbm.at[idx_v.at[j]], buf.at[p], gsems[p]).wait()

        def out_slice(j):
            return out_hbm.at[pl.ds(base + j * k, k)]

        def write_start(j, p):
            pltpu.async_copy(buf.at[p], out_slice(j), wsems[p])

        def write_wait(j, p):
            pltpu.make_async_copy(buf.at[p], out_slice(j), wsems[p]).wait()

        gather_start(0, 0)
        gather_start(1, 1)

        def loop_body(jj, carry):
            j0 = jj * 2
            for p in range(2):
                j = j0 + p
                gather_wait(j, p)
                write_start(j, p)

                @pl.when(j + 2 < nch)
                def _():
                    write_wait(j, p)
                    gather_start(j + 2, p)
            return carry

        lax.fori_loop(0, nch // 2, loop_body, 0)
        write_wait(nch - 2, 0)
        write_wait(nch - 1, 1)

    return body


def kernel(input_ids, word_embeddings):
    b, s = input_ids.shape
    v, h = word_embeddings.shape
    b_tot = b * s
    b_per_w = b_tot // _NW
    nch = b_per_w // _K
    idx = jnp.transpose(input_ids.astype(jnp.int32)).reshape(_NW, nch, _K)
    table = word_embeddings.astype(jnp.float32)
    out = _build_sc_gather(b_tot, v, h, _K, nch)(idx, table)
    return out.reshape(s, b, h)
